# Initial kernel scaffold; baseline (speedup 1.0000x reference)
import jax
import jax.numpy as jnp
from jax.experimental import pallas as pl


def kernel(x, w):
    raise NotImplementedError("write your pallas kernel here")


# speedup vs baseline: 5.2342x; 5.2342x over previous
import functools

import jax
import jax.numpy as jnp
from jax import lax
from jax.experimental import pallas as pl
from jax.experimental.pallas import tpu as pltpu
from jax.experimental.pallas import tpu_sc as plsc

BATCH = 128
C = 32768
K = 25
L = 16
NVREG = C // L
NW = 32
ROWS_PER_W = BATCH // NW
NBINS = 256


def _key_from_float(y):
    s = plsc.bitcast(y, jnp.int32)
    return s ^ (jnp.right_shift(s, 31) & jnp.int32(0x7FFFFFFF))


def _float_from_key(k):
    return plsc.bitcast(k ^ (jnp.right_shift(k, 31) & jnp.int32(0x7FFFFFFF)),
                        jnp.float32)


def _sc_body(x_hbm, w_hbm, out_hbm, w_v, row_v, cand_v, hist):
    wid = lax.axis_index("s") * 2 + lax.axis_index("c")
    lane = lax.iota(jnp.int32, L)
    ones = jnp.ones((L,), jnp.int32)

    pltpu.sync_copy(w_hbm, w_v)

    def zero_hist():
        def zh(b, _):
            hist[b] = jnp.zeros((L,), jnp.int32)
            return 0
        lax.fori_loop(0, NBINS, zh, 0)

    def select_byte(krem):
        def body(j, carry):
            acc, bsel, above = carry
            b = jnp.int32(NBINS - 1) - j
            cnt = jnp.sum(hist[b])
            found_now = jnp.logical_and(bsel < 0, acc + cnt >= krem)
            bsel = jnp.where(found_now, b, bsel)
            above = jnp.where(found_now, acc, above)
            return (acc + cnt, bsel, above)
        _, bsel, above = lax.fori_loop(
            0, NBINS, body, (jnp.int32(0), jnp.int32(-1), jnp.int32(0)))
        return bsel, krem - above

    def row_fn(r, _):
        row = wid * ROWS_PER_W + r
        pltpu.sync_copy(x_hbm.at[row], row_v)

        zero_hist()

        def p1(i, _):
            sl = pl.ds(i * L, L)
            y = row_v[sl] * w_v[sl]
            k = _key_from_float(y)
            row_v[sl] = plsc.bitcast(k, jnp.float32)
            byte = (jnp.right_shift(k, 24) & jnp.int32(0xFF)) ^ jnp.int32(0x80)
            plsc.addupdate_scatter(hist, [byte, lane], ones)
            return 0
        lax.fori_loop(0, NVREG, p1, 0)

        krem = jnp.int32(K)
        b1, krem = select_byte(krem)
        b1 = b1 ^ jnp.int32(0x80)

        ptr = jnp.zeros((L,), jnp.int32)
        def c1(i, ptr):
            kv = plsc.bitcast(row_v[pl.ds(i * L, L)], jnp.int32)
            keep = (jnp.right_shift(kv, 24) & jnp.int32(0xFF)) == b1
            ki = keep.astype(jnp.int32)
            idx = ptr + plsc.cumsum(ki) - ki
            plsc.store_scatter(cand_v, [idx], kv, mask=keep)
            return ptr + plsc.all_reduce_population_count(keep)
        ptr = lax.fori_loop(0, NVREG, c1, ptr)
        n = jnp.max(ptr)

        tau = jnp.left_shift(b1, 24)

        for shift in (16, 8):
            zero_hist()
            nv = (n + (L - 1)) // L

            def h2(i, _):
                base = i * L
                kv = cand_v[pl.ds(base, L)]
                valid = (lane + base) < n
                byte = jnp.right_shift(kv, shift) & jnp.int32(0xFF)
                plsc.addupdate_scatter(hist, [byte, lane], ones, mask=valid)
                return 0
            lax.fori_loop(0, nv, h2, 0)

            bl, krem = select_byte(krem)
            tau = tau | jnp.left_shift(bl, shift)

            ptr = jnp.zeros((L,), jnp.int32)
            def c2(i, ptr):
                base = i * L
                kv = cand_v[pl.ds(base, L)]
                valid = (lane + base) < n
                keep = jnp.logical_and(
                    (jnp.right_shift(kv, shift) & jnp.int32(0xFF)) == bl,
                    valid)
                ki = keep.astype(jnp.int32)
                idx = ptr + plsc.cumsum(ki) - ki
                plsc.store_scatter(cand_v, [idx], kv, mask=keep)
                return ptr + plsc.all_reduce_population_count(keep)
            ptr = lax.fori_loop(0, nv, c2, ptr)
            n = jnp.max(ptr)

        zero_hist()
        nv = (n + (L - 1)) // L

        def h4(i, _):
            base = i * L
            kv = cand_v[pl.ds(base, L)]
            valid = (lane + base) < n
            byte = kv & jnp.int32(0xFF)
            plsc.addupdate_scatter(hist, [byte, lane], ones, mask=valid)
            return 0
        lax.fori_loop(0, nv, h4, 0)

        b4, krem = select_byte(krem)
        tau = tau | b4

        def p3(i, _):
            sl = pl.ds(i * L, L)
            kv = plsc.bitcast(row_v[sl], jnp.int32)
            y = _float_from_key(kv)
            row_v[sl] = jnp.where(kv >= tau, y, jnp.float32(0.0))
            return 0
        lax.fori_loop(0, NVREG, p3, 0)

        pltpu.sync_copy(row_v, out_hbm.at[row])
        return 0

    lax.fori_loop(0, ROWS_PER_W, row_fn, 0)


@functools.partial(jax.jit)
def _sc_kfeature(x, w):
    mesh = plsc.VectorSubcoreMesh(core_axis_name="c", subcore_axis_name="s")
    f = functools.partial(
        pl.kernel,
        mesh=mesh,
        compiler_params=pltpu.CompilerParams(needs_layout_passes=False),
        out_type=jax.ShapeDtypeStruct((BATCH, C), jnp.float32),
        scratch_types=[
            pltpu.VMEM((C,), jnp.float32),
            pltpu.VMEM((C,), jnp.float32),
            pltpu.VMEM((C,), jnp.int32),
            pltpu.VMEM((NBINS, L), jnp.int32),
        ],
    )(_sc_body)
    return f(x, w)


def kernel(x, w):
    return _sc_kfeature(x, w)


# speedup vs baseline: 6.6427x; 1.2691x over previous
import functools

import jax
import jax.numpy as jnp
from jax import lax
from jax.experimental import pallas as pl
from jax.experimental.pallas import tpu as pltpu
from jax.experimental.pallas import tpu_sc as plsc

BATCH = 128
C = 32768
K = 25
L = 16
NVREG = C // L
NW = 32
ROWS_PER_W = BATCH // NW
NBINS = 256


def _key_from_float(y):
    s = plsc.bitcast(y, jnp.int32)
    return s ^ (jnp.right_shift(s, 31) & jnp.int32(0x7FFFFFFF))


def _float_from_key(k):
    return plsc.bitcast(k ^ (jnp.right_shift(k, 31) & jnp.int32(0x7FFFFFFF)),
                        jnp.float32)


def _sc_body(x_hbm, w_hbm, out_hbm, w_v, row_v, cand_v, hist):
    wid = lax.axis_index("s") * 2 + lax.axis_index("c")
    lane = lax.iota(jnp.int32, L)
    ones = jnp.ones((L,), jnp.int32)

    pltpu.sync_copy(w_hbm, w_v)

    def row_fn(r, _):
        row = wid * ROWS_PER_W + r
        pltpu.sync_copy(x_hbm.at[row], row_v)

        def zh(b, _):
            hist[b] = jnp.zeros((L,), jnp.int32)
            return 0
        lax.fori_loop(0, NBINS, zh, 0, unroll=8)

        def p1(i, _):
            sl = pl.ds(i * L, L)
            y = row_v[sl] * w_v[sl]
            k = _key_from_float(y)
            row_v[sl] = plsc.bitcast(k, jnp.float32)
            byte = (jnp.right_shift(k, 24) & jnp.int32(0xFF)) ^ jnp.int32(0x80)
            plsc.addupdate_scatter(hist, [byte, lane], ones)
            return 0
        lax.fori_loop(0, NVREG, p1, 0, unroll=8)

        def sb(j, carry):
            acc, bsel, above = carry
            b = jnp.int32(NBINS - 1) - j
            cnt = jnp.sum(hist[b])
            found_now = jnp.logical_and(bsel < 0, acc + cnt >= K)
            bsel = jnp.where(found_now, b, bsel)
            above = jnp.where(found_now, acc, above)
            return (acc + cnt, bsel, above)
        _, b1, above = lax.fori_loop(
            0, NBINS, sb, (jnp.int32(0), jnp.int32(-1), jnp.int32(0)),
            unroll=8)
        krem = jnp.int32(K) - above
        b1 = b1 ^ jnp.int32(0x80)

        def c1(i, ptr):
            kv = plsc.bitcast(row_v[pl.ds(i * L, L)], jnp.int32)
            keep = (jnp.right_shift(kv, 24) & jnp.int32(0xFF)) == b1
            ki = keep.astype(jnp.int32)
            idx = ptr + plsc.cumsum(ki) - ki
            plsc.store_scatter(cand_v, [idx], kv, mask=keep)
            return ptr + plsc.all_reduce_population_count(keep)
        ptr = lax.fori_loop(0, NVREG, c1, jnp.zeros((L,), jnp.int32),
                            unroll=4)
        n = jnp.max(ptr)

        nv = (n + (L - 1)) // L
        p = jnp.zeros((L,), jnp.int32)
        krem_v = jnp.zeros((L,), jnp.int32) + krem
        for b in range(23, -1, -1):
            patt = jnp.right_shift(p | jnp.int32(1 << b), b)

            def cb(i, c, patt=patt, b=b):
                v = cand_v[pl.ds(i * L, L)] & jnp.int32(0xFFFFFF)
                valid = (i * L + lane) < ptr
                m = jnp.logical_and(jnp.right_shift(v, b) == patt, valid)
                return c + plsc.all_reduce_population_count(m)
            c = lax.fori_loop(0, nv, cb, jnp.zeros((L,), jnp.int32))
            ge = c >= krem_v
            p = jnp.where(ge, p | jnp.int32(1 << b), p)
            krem_v = jnp.where(ge, krem_v, krem_v - c)

        tau = p | jnp.left_shift(b1, 24)

        def p3(i, _):
            sl = pl.ds(i * L, L)
            kv = plsc.bitcast(row_v[sl], jnp.int32)
            y = _float_from_key(kv)
            row_v[sl] = jnp.where(kv >= tau, y, jnp.float32(0.0))
            return 0
        lax.fori_loop(0, NVREG, p3, 0, unroll=8)

        pltpu.sync_copy(row_v, out_hbm.at[row])
        return 0

    lax.fori_loop(0, ROWS_PER_W, row_fn, 0)


@functools.partial(jax.jit)
def _sc_kfeature(x, w):
    mesh = plsc.VectorSubcoreMesh(core_axis_name="c", subcore_axis_name="s")
    f = functools.partial(
        pl.kernel,
        mesh=mesh,
        compiler_params=pltpu.CompilerParams(needs_layout_passes=False),
        out_type=jax.ShapeDtypeStruct((BATCH, C), jnp.float32),
        scratch_types=[
            pltpu.VMEM((C,), jnp.float32),
            pltpu.VMEM((C,), jnp.float32),
            pltpu.VMEM((C,), jnp.int32),
            pltpu.VMEM((NBINS, L), jnp.int32),
        ],
    )(_sc_body)
    return f(x, w)


def kernel(x, w):
    return _sc_kfeature(x, w)


# speedup vs baseline: 18.9024x; 2.8456x over previous
import functools

import jax
import jax.numpy as jnp
from jax import lax
from jax.experimental import pallas as pl
from jax.experimental.pallas import tpu as pltpu
from jax.experimental.pallas import tpu_sc as plsc

BATCH = 128
C = 32768
K = 25
L = 16
NVREG = C // L
NW = 32
ROWS_PER_W = BATCH // NW
NBINS = 256


def _key_from_float(y):
    s = plsc.bitcast(y, jnp.int32)
    return s ^ (jnp.right_shift(s, 31) & jnp.int32(0x7FFFFFFF))


def _float_from_key(k):
    return plsc.bitcast(k ^ (jnp.right_shift(k, 31) & jnp.int32(0x7FFFFFFF)),
                        jnp.float32)


def _sc_body(x_hbm, w_hbm, out_hbm, w_v, row_v, cand_v, hist):
    wid = lax.axis_index("s") * 2 + lax.axis_index("c")
    lane = lax.iota(jnp.int32, L)
    ones = jnp.ones((L,), jnp.int32)

    pltpu.sync_copy(w_hbm, w_v)

    def row_fn(r, _):
        row = wid * ROWS_PER_W + r
        pltpu.sync_copy(x_hbm.at[row], row_v)

        @plsc.parallel_loop(0, NBINS, unroll=8)
        def _(b):
            hist[b] = jnp.zeros((L,), jnp.int32)

        @plsc.parallel_loop(0, NVREG, unroll=8)
        def _(i):
            sl = pl.ds(i * L, L)
            y = row_v[sl] * w_v[sl]
            k = _key_from_float(y)
            row_v[sl] = plsc.bitcast(k, jnp.float32)
            byte = (jnp.right_shift(k, 24) & jnp.int32(0xFF)) ^ jnp.int32(0x80)
            plsc.addupdate_scatter(hist, [byte, lane], ones)

        @plsc.parallel_loop(
            0, NBINS, unroll=8,
            carry=(jnp.int32(0), jnp.int32(-1), jnp.int32(0)))
        def sb_out(j, carry):
            acc, bsel, above = carry
            b = jnp.int32(NBINS - 1) - j
            cnt = jnp.sum(hist[b])
            found_now = jnp.logical_and(bsel < 0, acc + cnt >= K)
            bsel = jnp.where(found_now, b, bsel)
            above = jnp.where(found_now, acc, above)
            return (acc + cnt, bsel, above)
        _, b1, above = sb_out
        krem = jnp.int32(K) - above
        b1 = b1 ^ jnp.int32(0x80)

        @plsc.parallel_loop(0, NVREG, unroll=4,
                            carry=jnp.zeros((L,), jnp.int32))
        def ptr(i, ptr):
            kv = plsc.bitcast(row_v[pl.ds(i * L, L)], jnp.int32)
            keep = (jnp.right_shift(kv, 24) & jnp.int32(0xFF)) == b1
            ki = keep.astype(jnp.int32)
            idx = ptr + plsc.cumsum(ki) - ki
            plsc.store_scatter(cand_v, [idx], kv, mask=keep)
            return ptr + plsc.all_reduce_population_count(keep)
        n = jnp.max(ptr)

        nv = (n + (L - 1)) // L
        p = jnp.zeros((L,), jnp.int32)
        krem_v = jnp.zeros((L,), jnp.int32) + krem
        for b in range(23, -1, -1):
            patt = jnp.right_shift(p | jnp.int32(1 << b), b)

            def cb(i, c, patt=patt, b=b):
                v = cand_v[pl.ds(i * L, L)] & jnp.int32(0xFFFFFF)
                valid = (i * L + lane) < ptr
                m = jnp.logical_and(jnp.right_shift(v, b) == patt, valid)
                return c + plsc.all_reduce_population_count(m)
            c = plsc.parallel_loop(
                0, nv, unroll=4, carry=jnp.zeros((L,), jnp.int32))(cb)
            ge = c >= krem_v
            p = jnp.where(ge, p | jnp.int32(1 << b), p)
            krem_v = jnp.where(ge, krem_v, krem_v - c)

        tau = p | jnp.left_shift(b1, 24)

        @plsc.parallel_loop(0, NVREG, unroll=8)
        def _(i):
            sl = pl.ds(i * L, L)
            kv = plsc.bitcast(row_v[sl], jnp.int32)
            y = _float_from_key(kv)
            row_v[sl] = jnp.where(kv >= tau, y, jnp.float32(0.0))

        pltpu.sync_copy(row_v, out_hbm.at[row])
        return 0

    lax.fori_loop(0, ROWS_PER_W, row_fn, 0)


@functools.partial(jax.jit)
def _sc_kfeature(x, w):
    mesh = plsc.VectorSubcoreMesh(core_axis_name="c", subcore_axis_name="s")
    f = functools.partial(
        pl.kernel,
        mesh=mesh,
        compiler_params=pltpu.CompilerParams(needs_layout_passes=False),
        out_type=jax.ShapeDtypeStruct((BATCH, C), jnp.float32),
        scratch_types=[
            pltpu.VMEM((C,), jnp.float32),
            pltpu.VMEM((C,), jnp.float32),
            pltpu.VMEM((C,), jnp.int32),
            pltpu.VMEM((NBINS, L), jnp.int32),
        ],
    )(_sc_body)
    return f(x, w)


def kernel(x, w):
    return _sc_kfeature(x, w)


# speedup vs baseline: 19.9334x; 1.0545x over previous
import functools

import jax
import jax.numpy as jnp
from jax import lax
from jax.experimental import pallas as pl
from jax.experimental.pallas import tpu as pltpu
from jax.experimental.pallas import tpu_sc as plsc

BATCH = 128
C = 32768
K = 25
L = 16
NVREG = C // L
NW = 32
ROWS_PER_W = BATCH // NW
NBINS = 256


def _key_from_float(y):
    s = plsc.bitcast(y, jnp.int32)
    return s ^ (jnp.right_shift(s, 31) & jnp.int32(0x7FFFFFFF))


def _float_from_key(k):
    return plsc.bitcast(k ^ (jnp.right_shift(k, 31) & jnp.int32(0x7FFFFFFF)),
                        jnp.float32)


NCH = 4
CH = C // NCH
CHV = CH // L


def _sc_body(x_hbm, w_hbm, out_hbm, w_v, row_v, cand_v, hist,
             sem_in0, sem_in1, sem_o0, sem_o1, sem_o2, sem_o3):
    wid = lax.axis_index("s") * 2 + lax.axis_index("c")
    lane = lax.iota(jnp.int32, L)
    ones = jnp.ones((L,), jnp.int32)
    sems_in = (sem_in0, sem_in1)
    sems_out = (sem_o0, sem_o1, sem_o2, sem_o3)

    pltpu.sync_copy(w_hbm, w_v)

    def row_fn(r, _):
        row = wid * ROWS_PER_W + r

        @plsc.parallel_loop(0, NBINS, unroll=8)
        def _(b):
            hist[b] = jnp.zeros((L,), jnp.int32)

        def issue_in(ch):
            return pltpu.async_copy(
                x_hbm.at[row, pl.ds(ch * CH, CH)],
                row_v.at[pl.ds(ch * CH, CH)],
                sems_in[ch % 2])
        in_h = issue_in(0)
        for ch in range(NCH):
            nxt = issue_in(ch + 1) if ch + 1 < NCH else None
            in_h.wait()
            in_h = nxt
            base = ch * CHV

            @plsc.parallel_loop(0, CHV, unroll=8)
            def _(i, base=base):
                sl = pl.ds((base + i) * L, L)
                y = row_v[sl] * w_v[sl]
                k = _key_from_float(y)
                row_v[sl] = plsc.bitcast(k, jnp.float32)
                byte = ((jnp.right_shift(k, 24) & jnp.int32(0xFF))
                        ^ jnp.int32(0x80))
                plsc.addupdate_scatter(hist, [byte, lane], ones)

        @plsc.parallel_loop(
            0, NBINS, unroll=8,
            carry=(jnp.int32(0), jnp.int32(-1), jnp.int32(0)))
        def sb_out(j, carry):
            acc, bsel, above = carry
            b = jnp.int32(NBINS - 1) - j
            cnt = jnp.sum(hist[b])
            found_now = jnp.logical_and(bsel < 0, acc + cnt >= K)
            bsel = jnp.where(found_now, b, bsel)
            above = jnp.where(found_now, acc, above)
            return (acc + cnt, bsel, above)
        _, b1, above = sb_out
        krem = jnp.int32(K) - above
        b1 = b1 ^ jnp.int32(0x80)

        @plsc.parallel_loop(0, NVREG, unroll=4,
                            carry=jnp.zeros((L,), jnp.int32))
        def ptr(i, ptr):
            kv = plsc.bitcast(row_v[pl.ds(i * L, L)], jnp.int32)
            keep = (jnp.right_shift(kv, 24) & jnp.int32(0xFF)) == b1
            ki = keep.astype(jnp.int32)
            idx = ptr + plsc.cumsum(ki) - ki
            plsc.store_scatter(cand_v, [idx], kv, mask=keep)
            return ptr + plsc.all_reduce_population_count(keep)
        n = jnp.max(ptr)

        nv = (n + (L - 1)) // L
        p = jnp.zeros((L,), jnp.int32)
        krem_v = jnp.zeros((L,), jnp.int32) + krem
        for b in range(23, -1, -1):
            patt = jnp.right_shift(p | jnp.int32(1 << b), b)

            def cb(i, c, patt=patt, b=b):
                v = cand_v[pl.ds(i * L, L)] & jnp.int32(0xFFFFFF)
                valid = (i * L + lane) < ptr
                m = jnp.logical_and(jnp.right_shift(v, b) == patt, valid)
                return c + plsc.all_reduce_population_count(m)
            c = plsc.parallel_loop(
                0, nv, unroll=4, carry=jnp.zeros((L,), jnp.int32))(cb)
            ge = c >= krem_v
            p = jnp.where(ge, p | jnp.int32(1 << b), p)
            krem_v = jnp.where(ge, krem_v, krem_v - c)

        tau = p | jnp.left_shift(b1, 24)

        out_h = []
        for ch in range(NCH):
            base = ch * CHV

            @plsc.parallel_loop(0, CHV, unroll=8)
            def _(i, base=base):
                sl = pl.ds((base + i) * L, L)
                kv = plsc.bitcast(row_v[sl], jnp.int32)
                y = _float_from_key(kv)
                row_v[sl] = jnp.where(kv >= tau, y, jnp.float32(0.0))

            out_h.append(pltpu.async_copy(
                row_v.at[pl.ds(ch * CH, CH)],
                out_hbm.at[row, pl.ds(ch * CH, CH)],
                sems_out[ch]))
        for h in out_h:
            h.wait()
        return 0

    lax.fori_loop(0, ROWS_PER_W, row_fn, 0)


@functools.partial(jax.jit)
def _sc_kfeature(x, w):
    mesh = plsc.VectorSubcoreMesh(core_axis_name="c", subcore_axis_name="s")
    f = functools.partial(
        pl.kernel,
        mesh=mesh,
        compiler_params=pltpu.CompilerParams(needs_layout_passes=False),
        out_type=jax.ShapeDtypeStruct((BATCH, C), jnp.float32),
        scratch_types=[
            pltpu.VMEM((C,), jnp.float32),
            pltpu.VMEM((C,), jnp.float32),
            pltpu.VMEM((C,), jnp.int32),
            pltpu.VMEM((NBINS, L), jnp.int32),
            pltpu.SemaphoreType.DMA,
            pltpu.SemaphoreType.DMA,
            pltpu.SemaphoreType.DMA,
            pltpu.SemaphoreType.DMA,
            pltpu.SemaphoreType.DMA,
            pltpu.SemaphoreType.DMA,
        ],
    )(_sc_body)
    return f(x, w)


def kernel(x, w):
    return _sc_kfeature(x, w)


# speedup vs baseline: 22.9210x; 1.1499x over previous
import functools

import jax
import jax.numpy as jnp
from jax import lax
from jax.experimental import pallas as pl
from jax.experimental.pallas import tpu as pltpu
from jax.experimental.pallas import tpu_sc as plsc

BATCH = 128
C = 32768
K = 25
L = 16
NVREG = C // L
NW = 32
ROWS_PER_W = BATCH // NW
NBINS = 256
NCH = 4
CH = C // NCH
CHV = CH // L


def _key_from_float(y):
    s = plsc.bitcast(y, jnp.int32)
    return s ^ (jnp.right_shift(s, 31) & jnp.int32(0x7FFFFFFF))


def _float_from_key(k):
    return plsc.bitcast(k ^ (jnp.right_shift(k, 31) & jnp.int32(0x7FFFFFFF)),
                        jnp.float32)


def _sc_body(x_hbm, w_hbm, out_hbm, w_v, row_v, cand_v, hist,
             sem_in0, sem_in1, sem_o0, sem_o1, sem_o2, sem_o3):
    wid = lax.axis_index("s") * 2 + lax.axis_index("c")
    lane = lax.iota(jnp.int32, L)
    ones = jnp.ones((L,), jnp.int32)
    sems_in = (sem_in0, sem_in1)
    sems_out = (sem_o0, sem_o1, sem_o2, sem_o3)

    pltpu.sync_copy(w_hbm, w_v)

    def row_fn(r, _):
        row = wid * ROWS_PER_W + r

        def issue_in(ch):
            return pltpu.async_copy(
                x_hbm.at[row, pl.ds(ch * CH, CH)],
                row_v.at[pl.ds(ch * CH, CH)],
                sems_in[ch % 2])

        in_h = issue_in(0)
        nxt = issue_in(1)
        in_h.wait()

        @plsc.parallel_loop(0, 32, unroll=8,
                            carry=jnp.full((L,), jnp.int32(-0x80000000)))
        def mx(i, mx):
            sl = pl.ds(i * (CHV // 32) * L, L)
            return jnp.maximum(mx, _key_from_float(row_v[sl] * w_v[sl]))
        mx_sorted, _ = plsc.sort_key_val(mx, mx, descending=True)
        cand_v[pl.ds(0, L)] = mx_sorted
        tau_est = plsc.load_gather(cand_v, [jnp.full((L,), 4, jnp.int32)])

        ptr = jnp.zeros((L,), jnp.int32)
        for ch in range(NCH):
            if ch > 0:
                in_h = nxt
                nxt = issue_in(ch + 1) if ch + 1 < NCH else None
                in_h.wait()
            base = ch * CHV

            @plsc.parallel_loop(0, CHV, unroll=8, carry=ptr)
            def ptr(i, ptr, base=base):
                sl = pl.ds((base + i) * L, L)
                k = _key_from_float(row_v[sl] * w_v[sl])
                row_v[sl] = plsc.bitcast(k, jnp.float32)
                keep = k >= tau_est
                ki = keep.astype(jnp.int32)
                idx = ptr + plsc.cumsum(ki) - ki
                plsc.store_scatter(cand_v, [idx], k, mask=keep)
                return ptr + plsc.all_reduce_population_count(keep)
        n = jnp.max(ptr)

        def search(nv, limit, krem0, cand_mask, cand_shift, nbits, prefix):
            p = jnp.zeros((L,), jnp.uint32)
            krem_v = jnp.zeros((L,), jnp.int32) + krem0
            for bi in range(nbits - 1, -1, -1):
                patt = jnp.right_shift(p | jnp.uint32(1 << bi),
                                       jnp.uint32(bi))

                def cb(i, c, patt=patt, bi=bi):
                    kv = cand_v[pl.ds(i * L, L)]
                    v = (plsc.bitcast(kv, jnp.uint32) ^ cand_mask) \
                        & cand_shift
                    m = jnp.logical_and(
                        jnp.right_shift(v, jnp.uint32(bi)) == patt,
                        (i * L + lane) < limit)
                    return c + plsc.all_reduce_population_count(m)
                c = plsc.parallel_loop(
                    0, nv, unroll=4, carry=jnp.zeros((L,), jnp.int32))(cb)
                ge = c >= krem_v
                p = jnp.where(ge, p | jnp.uint32(1 << bi), p)
                krem_v = jnp.where(ge, krem_v, krem_v - c)
            return prefix | plsc.bitcast(p ^ cand_mask, jnp.int32)

        def good_path(n, ptr):
            nv = (n + (L - 1)) // L
            return search(nv, ptr, jnp.int32(K),
                          jnp.uint32(0x80000000), jnp.uint32(0xFFFFFFFF),
                          32, jnp.zeros((L,), jnp.int32))

        def fallback_path(n, ptr):
            @plsc.parallel_loop(0, NBINS, unroll=8)
            def _(b):
                hist[b] = jnp.zeros((L,), jnp.int32)

            @plsc.parallel_loop(0, NVREG, unroll=8)
            def _(i):
                k = plsc.bitcast(row_v[pl.ds(i * L, L)], jnp.int32)
                byte = ((jnp.right_shift(k, 24) & jnp.int32(0xFF))
                        ^ jnp.int32(0x80))
                plsc.addupdate_scatter(hist, [byte, lane], ones)

            @plsc.parallel_loop(
                0, NBINS, unroll=8,
                carry=(jnp.int32(0), jnp.int32(-1), jnp.int32(0)))
            def sb_out(j, carry):
                acc, bsel, above = carry
                b = jnp.int32(NBINS - 1) - j
                cnt = jnp.sum(hist[b])
                found_now = jnp.logical_and(bsel < 0, acc + cnt >= K)
                bsel = jnp.where(found_now, b, bsel)
                above = jnp.where(found_now, acc, above)
                return (acc + cnt, bsel, above)
            _, b1, above = sb_out
            krem = jnp.int32(K) - above
            b1 = b1 ^ jnp.int32(0x80)

            @plsc.parallel_loop(0, NVREG, unroll=4,
                                carry=jnp.zeros((L,), jnp.int32))
            def ptr2(i, ptr2):
                kv = plsc.bitcast(row_v[pl.ds(i * L, L)], jnp.int32)
                keep = (jnp.right_shift(kv, 24) & jnp.int32(0xFF)) == b1
                ki = keep.astype(jnp.int32)
                idx = ptr2 + plsc.cumsum(ki) - ki
                plsc.store_scatter(cand_v, [idx], kv, mask=keep)
                return ptr2 + plsc.all_reduce_population_count(keep)
            n2 = jnp.max(ptr2)
            nv2 = (n2 + (L - 1)) // L
            return search(nv2, ptr2, krem,
                          jnp.uint32(0), jnp.uint32(0xFFFFFF),
                          24, jnp.left_shift(jnp.zeros((L,), jnp.int32) + b1,
                                             24))

        tau = lax.cond(n >= K, good_path, fallback_path, n, ptr)

        out_h = []
        for ch in range(NCH):
            base = ch * CHV

            @plsc.parallel_loop(0, CHV, unroll=8)
            def _(i, base=base):
                sl = pl.ds((base + i) * L, L)
                kv = plsc.bitcast(row_v[sl], jnp.int32)
                y = _float_from_key(kv)
                row_v[sl] = jnp.where(kv >= tau, y, jnp.float32(0.0))

            out_h.append(pltpu.async_copy(
                row_v.at[pl.ds(ch * CH, CH)],
                out_hbm.at[row, pl.ds(ch * CH, CH)],
                sems_out[ch]))
        for h in out_h:
            h.wait()
        return 0

    lax.fori_loop(0, ROWS_PER_W, row_fn, 0)


@functools.partial(jax.jit)
def _sc_kfeature(x, w):
    mesh = plsc.VectorSubcoreMesh(core_axis_name="c", subcore_axis_name="s")
    f = functools.partial(
        pl.kernel,
        mesh=mesh,
        compiler_params=pltpu.CompilerParams(needs_layout_passes=False),
        out_type=jax.ShapeDtypeStruct((BATCH, C), jnp.float32),
        scratch_types=[
            pltpu.VMEM((C,), jnp.float32),
            pltpu.VMEM((C,), jnp.float32),
            pltpu.VMEM((C,), jnp.int32),
            pltpu.VMEM((NBINS, L), jnp.int32),
            pltpu.SemaphoreType.DMA,
            pltpu.SemaphoreType.DMA,
            pltpu.SemaphoreType.DMA,
            pltpu.SemaphoreType.DMA,
            pltpu.SemaphoreType.DMA,
            pltpu.SemaphoreType.DMA,
        ],
    )(_sc_body)
    return f(x, w)


def kernel(x, w):
    return _sc_kfeature(x, w)
